# baseline (speedup 1.0000x reference)
import functools

import jax
import jax.numpy as jnp
import numpy as np
from jax.experimental import pallas as pl
from jax.experimental.pallas import tpu as pltpu


MASK_PROB = 0.65
MASK_LENGTH = 10
MIN_MASKS = 2


@functools.lru_cache(maxsize=None)
def _static_mask(B: int, T: int):
    rng = np.random.RandomState(0)
    mask = np.zeros((B, T), dtype=bool)
    for i in range(B):
        seq_len = T
        num_mask = int(MASK_PROB * seq_len / float(MASK_LENGTH) + rng.rand())
        num_mask = max(MIN_MASKS, num_mask)
        min_len = MASK_LENGTH
        if seq_len - min_len <= num_mask:
            min_len = seq_len - num_mask - 1
        mask_idc = rng.choice(seq_len - min_len, num_mask, replace=False)
        mask_idc = np.asarray(
            [mask_idc[j] + offset for j in range(len(mask_idc)) for offset in range(MASK_LENGTH)]
        )
        mask[i, mask_idc] = True
    return mask


def _select_kernel(cmr_ref, pmr_ref, x_ref, emb_ref, out_ref, mask_ref):
    m_i32 = (cmr_ref[0, 0] != 0) & (pmr_ref[0, 0] == 0)
    mask_ref[0, 0] = m_i32
    m_col = m_i32.astype(jnp.int32).reshape(-1, 1) != 0
    out_ref[0] = jnp.where(m_col, emb_ref[...], x_ref[0])


def kernel(tensor, padding_mask, mask_emb):
    B, T, D = tensor.shape
    TT = 4096
    nt = T // TT
    cm = jnp.asarray(_static_mask(B, T).astype(np.int32))
    pm = padding_mask.astype(jnp.int32)
    cm_row = cm.reshape(B, nt, 1, TT)
    pm_row = pm.reshape(B, nt, 1, TT)
    emb2 = mask_emb.reshape(1, D)

    row_spec = pl.BlockSpec((1, 1, 1, TT), lambda i, j: (i, j, 0, 0))
    out, mask_i32 = pl.pallas_call(
        _select_kernel,
        grid=(B, nt),
        in_specs=[
            row_spec,
            row_spec,
            pl.BlockSpec((1, TT, D), lambda i, j: (i, j, 0)),
            pl.BlockSpec((1, D), lambda i, j: (0, 0)),
        ],
        out_specs=[
            pl.BlockSpec((1, TT, D), lambda i, j: (i, j, 0)),
            row_spec,
        ],
        out_shape=[
            jax.ShapeDtypeStruct((B, T, D), tensor.dtype),
            jax.ShapeDtypeStruct((B, nt, 1, TT), jnp.bool_),
        ],
        compiler_params=pltpu.CompilerParams(
            dimension_semantics=("parallel", "parallel"),
        ),
    )(cm_row, pm_row, tensor, emb2)

    return out, mask_i32.reshape(B, T)


# speedup vs baseline: 1.0087x; 1.0087x over previous
import functools

import jax
import jax.numpy as jnp
import numpy as np
from jax.experimental import pallas as pl
from jax.experimental.pallas import tpu as pltpu


MASK_PROB = 0.65
MASK_LENGTH = 10
MIN_MASKS = 2


@functools.lru_cache(maxsize=None)
def _static_mask(B: int, T: int):
    rng = np.random.RandomState(0)
    mask = np.zeros((B, T), dtype=bool)
    for i in range(B):
        seq_len = T
        num_mask = int(MASK_PROB * seq_len / float(MASK_LENGTH) + rng.rand())
        num_mask = max(MIN_MASKS, num_mask)
        min_len = MASK_LENGTH
        if seq_len - min_len <= num_mask:
            min_len = seq_len - num_mask - 1
        mask_idc = rng.choice(seq_len - min_len, num_mask, replace=False)
        mask_idc = np.asarray(
            [mask_idc[j] + offset for j in range(len(mask_idc)) for offset in range(MASK_LENGTH)]
        )
        mask[i, mask_idc] = True
    return mask


def _select_kernel(cm_ref, pm_ref, x_ref, emb_ref, out_ref, mask_ref):
    i = pl.program_id(0)

    @pl.when(i == 0)
    def _():
        mask_ref[...] = (cm_ref[...] != 0) & (pm_ref[...] == 0)

    m_row = (cm_ref[pl.ds(i, 1), :] != 0) & (pm_ref[pl.ds(i, 1), :] == 0)
    m_col = m_row.astype(jnp.int32).reshape(-1, 1) != 0
    out_ref[0] = jnp.where(m_col, emb_ref[...], x_ref[0])


def kernel(tensor, padding_mask, mask_emb):
    B, T, D = tensor.shape
    cm = jnp.asarray(_static_mask(B, T).astype(np.int32))
    pm = padding_mask.astype(jnp.int32)
    emb2 = mask_emb.reshape(1, D)

    out, mask = pl.pallas_call(
        _select_kernel,
        grid=(B,),
        in_specs=[
            pl.BlockSpec(memory_space=pltpu.VMEM),
            pl.BlockSpec(memory_space=pltpu.VMEM),
            pl.BlockSpec((1, T, D), lambda i: (i, 0, 0)),
            pl.BlockSpec(memory_space=pltpu.VMEM),
        ],
        out_specs=[
            pl.BlockSpec((1, T, D), lambda i: (i, 0, 0)),
            pl.BlockSpec(memory_space=pltpu.VMEM),
        ],
        out_shape=[
            jax.ShapeDtypeStruct((B, T, D), tensor.dtype),
            jax.ShapeDtypeStruct((B, T), jnp.bool_),
        ],
        compiler_params=pltpu.CompilerParams(
            dimension_semantics=("arbitrary",),
        ),
    )(cm, pm, tensor, emb2)

    return out, mask
